# baseline (speedup 1.0000x reference)
import functools

import jax
import jax.numpy as jnp
from jax import lax
from jax.experimental import pallas as pl
from jax.experimental.pallas import tpu as pltpu
from jax.experimental.pallas import tpu_sc as plsc


def _stage1_slab_prefix(dag, W1, b1r, W2bd, b2t, tri, sred, wtri, Rp):
    N, D = dag.shape
    H1 = W1.shape[1]
    N8 = N // 8
    steps = N8 // Rp

    def body(x0, x1, x2, x3, x4, x5, x6, x7, w1_ref, b1_ref, w2_ref, b2_ref,
             tri_ref, sred_ref, wtri_ref, p_ref, carry_ref):
        g = pl.program_id(0)

        @pl.when(g == 0)
        def _():
            carry_ref[...] = jnp.zeros_like(carry_ref)

        carry = carry_ref[...]

        @pl.when(g < steps)
        def _():
            w1 = w1_ref[...]
            b1 = b1_ref[...]
            h1s = [
                jnp.maximum(
                    jnp.dot(x_ref[...], w1,
                            preferred_element_type=jnp.float32) + b1, 0.0)
                for x_ref in (x0, x1, x2, x3, x4, x5, x6, x7)
            ]
            h1cat = jnp.concatenate(h1s, axis=1)
            h2 = jnp.maximum(
                jnp.dot(h1cat, w2_ref[...],
                        preferred_element_type=jnp.float32) + b2_ref[...],
                0.0)
            hloc = jnp.dot(tri_ref[...], h2,
                           preferred_element_type=jnp.float32,
                           precision=lax.Precision.HIGHEST)
            p_ref[...] = hloc + carry
            carry_ref[...] = (carry + hloc[Rp - 1:Rp, :]
                              + h2[Rp - 1:Rp, :])

        @pl.when(g == steps)
        def _():
            tot = jnp.dot(carry, sred_ref[...],
                          preferred_element_type=jnp.float32,
                          precision=lax.Precision.HIGHEST)
            off = jnp.dot(carry, wtri_ref[...],
                          preferred_element_type=jnp.float32,
                          precision=lax.Precision.HIGHEST)
            row = lax.broadcasted_iota(jnp.int32, (Rp, 128), 0)
            p_ref[...] = jnp.where(row == 0, tot,
                                   jnp.where(row == 1, off, 0.0))

    def xmap(k):
        return lambda g: (jnp.minimum(g, steps - 1) + k * steps, 0)

    return pl.pallas_call(
        body,
        grid=(steps + 1,),
        in_specs=[pl.BlockSpec((Rp, D), xmap(k)) for k in range(8)] + [
            pl.BlockSpec((D, H1), lambda g: (0, 0)),
            pl.BlockSpec((1, H1), lambda g: (0, 0)),
            pl.BlockSpec((128, 128), lambda g: (0, 0)),
            pl.BlockSpec((1, 128), lambda g: (0, 0)),
            pl.BlockSpec((Rp, Rp), lambda g: (0, 0)),
            pl.BlockSpec((128, 128), lambda g: (0, 0)),
            pl.BlockSpec((128, 128), lambda g: (0, 0)),
        ],
        out_specs=pl.BlockSpec((Rp, 128), lambda g: (g, 0)),
        out_shape=jax.ShapeDtypeStruct((N8 + Rp, 128), jnp.float32),
        scratch_shapes=[pltpu.VMEM((1, 128), jnp.float32)],
    )(dag, dag, dag, dag, dag, dag, dag, dag,
      W1, b1r, W2bd, b2t, tri, sred, wtri)


def _make_consts(Rp):
    r2 = lax.broadcasted_iota(jnp.int32, (128, 128), 0)
    c2 = lax.broadcasted_iota(jnp.int32, (128, 128), 1)
    S = (r2 % 16 == c2).astype(jnp.float32)[:, :16]
    sred = (r2 % 16 == c2 % 16).astype(jnp.float32)
    wtri = ((r2 % 16 == c2 % 16) & (r2 // 16 < c2 // 16)).astype(jnp.float32)
    rg = lax.broadcasted_iota(jnp.int32, (Rp, Rp), 0)
    cg = lax.broadcasted_iota(jnp.int32, (Rp, Rp), 1)
    tri = (cg < rg).astype(jnp.float32)
    return tri, S, sred, wtri


def _stage2_gather(Pp, idx_pad, b_per_w, n8):
    Bpad = idx_pad.shape[0]
    nseg16 = b_per_w // 16
    mesh = plsc.VectorSubcoreMesh(core_axis_name="c", subcore_axis_name="s")
    info = plsc.get_sparse_core_info()
    NC = info.num_cores

    @functools.partial(
        pl.kernel,
        mesh=mesh,
        out_type=jax.ShapeDtypeStruct((Bpad, 128), jnp.float32),
        scratch_types=[
            pltpu.VMEM((b_per_w,), jnp.int32),
            pltpu.VMEM((b_per_w, 128), jnp.float32),
            pltpu.SemaphoreType.DMA,
        ],
    )
    def gather_k(p_hbm, idx_hbm, out_hbm, idx_v, rows_v, sem):
        wid = lax.axis_index("s") * NC + lax.axis_index("c")
        base = wid * b_per_w
        pltpu.sync_copy(idx_hbm.at[pl.ds(base, b_per_w)], idx_v)
        for i in range(nseg16):
            sl = pl.ds(i * 16, 16)
            v = idx_v[sl]
            idx_v[sl] = jnp.bitwise_or(
                jnp.bitwise_and(v, n8 - 1),
                jnp.bitwise_and(lax.shift_right_logical(v, 3), n8))
        pltpu.async_copy(p_hbm.at[idx_v], rows_v, sem).wait()
        pltpu.sync_copy(rows_v, out_hbm.at[pl.ds(base, b_per_w)])

    return gather_k(Pp, idx_pad)


def _stage3_output(Gp, off, ptr_i, S, W3p, b3r, Bseg, n8_log2):
    D = W3p.shape[1]
    Bp1 = Bseg + 1

    def body(g_ref, off_ref, pi_ref, s_ref, w3_ref, b3_ref, o_ref):
        gp = g_ref[...][:Bp1, :]
        pi = pi_ref[...]
        rem = jnp.bitwise_and(
            lax.shift_right_logical(pi, n8_log2), 7)
        win = lax.broadcasted_iota(jnp.int32, (Bp1, 128), 1) // 16
        masked = jnp.where(win == rem, gp + off_ref[...], 0.0)
        ext = jnp.dot(masked, s_ref[...],
                      preferred_element_type=jnp.float32,
                      precision=lax.Precision.HIGHEST)
        d = ext[1:, :] - ext[:Bseg, :]
        cnt = (pi[1:, :] - pi[:Bseg, :]).astype(jnp.float32)
        o_ref[...] = (
            jnp.dot(d, w3_ref[...], preferred_element_type=jnp.float32,
                    precision=lax.Precision.HIGHEST)
            + cnt * b3_ref[...])

    return pl.pallas_call(
        body,
        out_shape=jax.ShapeDtypeStruct((Bseg, D), jnp.float32),
    )(Gp, off, ptr_i, S, W3p, b3r)


def kernel(dag_summaries, obs_ptr, W1, b1, W2, b2, W3, b3):
    N, D = dag_summaries.shape
    H1 = W1.shape[1]
    H2 = W2.shape[1]
    Bseg = obs_ptr.shape[0] - 1
    Rp = 1024
    N8 = N // 8
    n8_log2 = N8.bit_length() - 1

    ptr = obs_ptr.astype(jnp.int32)

    W2p = jnp.zeros((H1, 16), jnp.float32).at[:, :H2].set(W2)
    b2p = jnp.zeros((1, 16), jnp.float32).at[0, :H2].set(b2)
    W3p = jnp.zeros((16, D), jnp.float32).at[:H2, :].set(W3)
    b1r = b1.reshape(1, H1)
    b3r = b3.reshape(1, D)
    W2bd = jnp.kron(jnp.eye(8, dtype=jnp.float32), W2p)
    b2t = jnp.tile(b2p, (1, 8))

    tri, S, sred, wtri = _make_consts(Rp)
    Pp = _stage1_slab_prefix(dag_summaries, W1, b1r, W2bd, b2t,
                             tri, sred, wtri, Rp)

    info = plsc.get_sparse_core_info()
    NW = info.num_cores * info.num_subcores
    chunk = 16 * NW
    Bpad = ((Bseg + 1 + chunk - 1) // chunk) * chunk
    idx_pad = jnp.zeros((Bpad,), jnp.int32).at[:Bseg + 1].set(ptr)
    Gp = _stage2_gather(Pp, idx_pad, Bpad // NW, N8)

    off = lax.slice(Pp, (N8 + 1, 0), (N8 + 2, 128))
    ptr_i = ptr.reshape(Bseg + 1, 1)
    return _stage3_output(Gp, off, ptr_i, S, W3p, b3r, Bseg, n8_log2)


# speedup vs baseline: 1.5439x; 1.5439x over previous
import functools

import jax
import jax.numpy as jnp
from jax import lax
from jax.experimental import pallas as pl
from jax.experimental.pallas import tpu as pltpu
from jax.experimental.pallas import tpu_sc as plsc


def _stage1_slab_prefix(dag, W1, b1r, W2bd, b2t, tri, sred, wtri, Rp):
    N, D = dag.shape
    H1 = W1.shape[1]
    N8 = N // 8
    steps = N8 // Rp

    def body(x0, x1, x2, x3, x4, x5, x6, x7, w1_ref, b1_ref, w2_ref, b2_ref,
             tri_ref, sred_ref, wtri_ref, p_ref, carry_ref):
        g = pl.program_id(0)

        @pl.when(g == 0)
        def _():
            carry_ref[...] = jnp.zeros_like(carry_ref)

        carry = carry_ref[...]

        @pl.when(g < steps)
        def _():
            w1 = w1_ref[...]
            b1 = b1_ref[...]
            h1s = [
                jnp.maximum(
                    jnp.dot(x_ref[...], w1,
                            preferred_element_type=jnp.float32) + b1, 0.0)
                for x_ref in (x0, x1, x2, x3, x4, x5, x6, x7)
            ]
            h1cat = jnp.concatenate(h1s, axis=1)
            h2 = jnp.maximum(
                jnp.dot(h1cat, w2_ref[...],
                        preferred_element_type=jnp.float32) + b2_ref[...],
                0.0)
            tri = tri_ref[...]
            o = carry
            for c in range(Rp // 128):
                h2c = h2[c * 128:(c + 1) * 128, :]
                hl = jnp.dot(tri, h2c,
                             preferred_element_type=jnp.float32,
                             precision=lax.Precision.HIGHEST)
                p_ref[c * 128:(c + 1) * 128, :] = hl + o
                o = o + hl[127:128, :] + h2c[127:128, :]
            carry_ref[...] = o

        @pl.when(g == steps)
        def _():
            tot = jnp.dot(carry, sred_ref[...],
                          preferred_element_type=jnp.float32,
                          precision=lax.Precision.HIGHEST)
            off = jnp.dot(carry, wtri_ref[...],
                          preferred_element_type=jnp.float32,
                          precision=lax.Precision.HIGHEST)
            row = lax.broadcasted_iota(jnp.int32, (Rp, 128), 0)
            p_ref[...] = jnp.where(row == 0, tot,
                                   jnp.where(row == 1, off, 0.0))

    def xmap(k):
        return lambda g: (jnp.minimum(g, steps - 1) + k * steps, 0)

    return pl.pallas_call(
        body,
        grid=(steps + 1,),
        in_specs=[pl.BlockSpec((Rp, D), xmap(k)) for k in range(8)] + [
            pl.BlockSpec((D, H1), lambda g: (0, 0)),
            pl.BlockSpec((1, H1), lambda g: (0, 0)),
            pl.BlockSpec((128, 128), lambda g: (0, 0)),
            pl.BlockSpec((1, 128), lambda g: (0, 0)),
            pl.BlockSpec((128, 128), lambda g: (0, 0)),
            pl.BlockSpec((128, 128), lambda g: (0, 0)),
            pl.BlockSpec((128, 128), lambda g: (0, 0)),
        ],
        out_specs=pl.BlockSpec((Rp, 128), lambda g: (g, 0)),
        out_shape=jax.ShapeDtypeStruct((N8 + Rp, 128), jnp.float32),
        scratch_shapes=[pltpu.VMEM((1, 128), jnp.float32)],
    )(dag, dag, dag, dag, dag, dag, dag, dag,
      W1, b1r, W2bd, b2t, tri, sred, wtri)


def _make_consts(Rp):
    r2 = lax.broadcasted_iota(jnp.int32, (128, 128), 0)
    c2 = lax.broadcasted_iota(jnp.int32, (128, 128), 1)
    S = (r2 % 16 == c2).astype(jnp.float32)[:, :16]
    sred = (r2 % 16 == c2 % 16).astype(jnp.float32)
    wtri = ((r2 % 16 == c2 % 16) & (r2 // 16 < c2 // 16)).astype(jnp.float32)
    tri = (c2 < r2).astype(jnp.float32)
    return tri, S, sred, wtri


def _stage2_gather(Pp, idx_pad, b_per_w, n8):
    Bpad = idx_pad.shape[0]
    nseg16 = b_per_w // 16
    mesh = plsc.VectorSubcoreMesh(core_axis_name="c", subcore_axis_name="s")
    info = plsc.get_sparse_core_info()
    NC = info.num_cores

    @functools.partial(
        pl.kernel,
        mesh=mesh,
        out_type=jax.ShapeDtypeStruct((Bpad, 128), jnp.float32),
        scratch_types=[
            pltpu.VMEM((b_per_w,), jnp.int32),
            pltpu.VMEM((b_per_w, 128), jnp.float32),
            pltpu.SemaphoreType.DMA,
        ],
    )
    def gather_k(p_hbm, idx_hbm, out_hbm, idx_v, rows_v, sem):
        wid = lax.axis_index("s") * NC + lax.axis_index("c")
        base = wid * b_per_w
        pltpu.sync_copy(idx_hbm.at[pl.ds(base, b_per_w)], idx_v)
        for i in range(nseg16):
            sl = pl.ds(i * 16, 16)
            v = idx_v[sl]
            idx_v[sl] = jnp.bitwise_or(
                jnp.bitwise_and(v, n8 - 1),
                jnp.bitwise_and(lax.shift_right_logical(v, 3), n8))
        pltpu.async_copy(p_hbm.at[idx_v], rows_v, sem).wait()
        pltpu.sync_copy(rows_v, out_hbm.at[pl.ds(base, b_per_w)])

    return gather_k(Pp, idx_pad)


def _stage3_output(Gp, off, ptr_i, S, W3p, b3r, Bseg, n8_log2):
    D = W3p.shape[1]
    Bp1 = Bseg + 1

    def body(g_ref, off_ref, pi_ref, s_ref, w3_ref, b3_ref, o_ref):
        gp = g_ref[...][:Bp1, :]
        pi = pi_ref[...]
        rem = jnp.bitwise_and(
            lax.shift_right_logical(pi, n8_log2), 7)
        win = lax.broadcasted_iota(jnp.int32, (Bp1, 128), 1) // 16
        masked = jnp.where(win == rem, gp + off_ref[...], 0.0)
        ext = jnp.dot(masked, s_ref[...],
                      preferred_element_type=jnp.float32,
                      precision=lax.Precision.HIGHEST)
        d = ext[1:, :] - ext[:Bseg, :]
        cnt = (pi[1:, :] - pi[:Bseg, :]).astype(jnp.float32)
        o_ref[...] = (
            jnp.dot(d, w3_ref[...], preferred_element_type=jnp.float32,
                    precision=lax.Precision.HIGHEST)
            + cnt * b3_ref[...])

    return pl.pallas_call(
        body,
        out_shape=jax.ShapeDtypeStruct((Bseg, D), jnp.float32),
    )(Gp, off, ptr_i, S, W3p, b3r)


def kernel(dag_summaries, obs_ptr, W1, b1, W2, b2, W3, b3):
    N, D = dag_summaries.shape
    H1 = W1.shape[1]
    H2 = W2.shape[1]
    Bseg = obs_ptr.shape[0] - 1
    Rp = 512
    N8 = N // 8
    n8_log2 = N8.bit_length() - 1

    ptr = obs_ptr.astype(jnp.int32)

    W2p = jnp.zeros((H1, 16), jnp.float32).at[:, :H2].set(W2)
    b2p = jnp.zeros((1, 16), jnp.float32).at[0, :H2].set(b2)
    W3p = jnp.zeros((16, D), jnp.float32).at[:H2, :].set(W3)
    b1r = b1.reshape(1, H1)
    b3r = b3.reshape(1, D)
    W2bd = jnp.kron(jnp.eye(8, dtype=jnp.float32), W2p)
    b2t = jnp.tile(b2p, (1, 8))

    tri, S, sred, wtri = _make_consts(Rp)
    Pp = _stage1_slab_prefix(dag_summaries, W1, b1r, W2bd, b2t,
                             tri, sred, wtri, Rp)

    info = plsc.get_sparse_core_info()
    NW = info.num_cores * info.num_subcores
    chunk = 16 * NW
    Bpad = ((Bseg + 1 + chunk - 1) // chunk) * chunk
    idx_pad = jnp.zeros((Bpad,), jnp.int32).at[:Bseg + 1].set(ptr)
    Gp = _stage2_gather(Pp, idx_pad, Bpad // NW, N8)

    off = lax.slice(Pp, (N8 + 1, 0), (N8 + 2, 128))
    ptr_i = ptr.reshape(Bseg + 1, 1)
    return _stage3_output(Gp, off, ptr_i, S, W3p, b3r, Bseg, n8_log2)


# speedup vs baseline: 1.9015x; 1.2316x over previous
import functools

import jax
import jax.numpy as jnp
from jax import lax
from jax.experimental import pallas as pl
from jax.experimental.pallas import tpu as pltpu
from jax.experimental.pallas import tpu_sc as plsc


def _stage1_slab_prefix(dag, W1, b1r, W2bd, b2t, tri, sred, wtri, Rp):
    N, D = dag.shape
    H1 = W1.shape[1]
    N8 = N // 8
    steps = N8 // Rp

    def body(x0, x1, x2, x3, x4, x5, x6, x7, w1_ref, b1_ref, w2_ref, b2_ref,
             tri_ref, sred_ref, wtri_ref, p_ref, carry_ref):
        g = pl.program_id(0)

        @pl.when(g == 0)
        def _():
            carry_ref[...] = jnp.zeros_like(carry_ref)

        carry = carry_ref[...]

        @pl.when(g < steps)
        def _():
            w1 = w1_ref[...]
            b1 = b1_ref[...]
            h1s = [
                jnp.maximum(
                    jnp.dot(x_ref[...], w1,
                            preferred_element_type=jnp.float32) + b1, 0.0)
                for x_ref in (x0, x1, x2, x3, x4, x5, x6, x7)
            ]
            h1cat = jnp.concatenate(h1s, axis=1)
            h2 = jnp.maximum(
                jnp.dot(h1cat, w2_ref[...],
                        preferred_element_type=jnp.float32) + b2_ref[...],
                0.0)
            tri = tri_ref[...]
            o = carry
            for c in range(Rp // 128):
                h2c = h2[c * 128:(c + 1) * 128, :]
                hl = jnp.dot(tri, h2c,
                             preferred_element_type=jnp.float32,
                             precision=lax.Precision.HIGHEST)
                p_ref[c * 128:(c + 1) * 128, :] = hl + o
                o = o + hl[127:128, :] + h2c[127:128, :]
            carry_ref[...] = o

        @pl.when(g == steps)
        def _():
            tot = jnp.dot(carry, sred_ref[...],
                          preferred_element_type=jnp.float32,
                          precision=lax.Precision.HIGHEST)
            off = jnp.dot(carry, wtri_ref[...],
                          preferred_element_type=jnp.float32,
                          precision=lax.Precision.HIGHEST)
            row = lax.broadcasted_iota(jnp.int32, (Rp, 128), 0)
            p_ref[...] = jnp.where(row == 0, tot,
                                   jnp.where(row == 1, off, 0.0))

    def xmap(k):
        return lambda g: (jnp.minimum(g, steps - 1) + k * steps, 0)

    return pl.pallas_call(
        body,
        grid=(steps + 1,),
        in_specs=[pl.BlockSpec((Rp, D), xmap(k)) for k in range(8)] + [
            pl.BlockSpec((D, H1), lambda g: (0, 0)),
            pl.BlockSpec((1, H1), lambda g: (0, 0)),
            pl.BlockSpec((128, 128), lambda g: (0, 0)),
            pl.BlockSpec((1, 128), lambda g: (0, 0)),
            pl.BlockSpec((128, 128), lambda g: (0, 0)),
            pl.BlockSpec((128, 128), lambda g: (0, 0)),
            pl.BlockSpec((128, 128), lambda g: (0, 0)),
        ],
        out_specs=pl.BlockSpec((Rp, 128), lambda g: (g, 0)),
        out_shape=jax.ShapeDtypeStruct((N8 + Rp, 128), jnp.float32),
        scratch_shapes=[pltpu.VMEM((1, 128), jnp.float32)],
    )(dag, dag, dag, dag, dag, dag, dag, dag,
      W1, b1r, W2bd, b2t, tri, sred, wtri)


def _make_consts(Rp):
    r2 = lax.broadcasted_iota(jnp.int32, (128, 128), 0)
    c2 = lax.broadcasted_iota(jnp.int32, (128, 128), 1)
    S = (r2 % 16 == c2).astype(jnp.float32)[:, :16]
    sred = (r2 % 16 == c2 % 16).astype(jnp.float32)
    wtri = ((r2 % 16 == c2 % 16) & (r2 // 16 < c2 // 16)).astype(jnp.float32)
    tri = (c2 < r2).astype(jnp.float32)
    return tri, S, sred, wtri


def _stage2_gather(Pp, idx_pad, b_per_w, n8):
    Bpad = idx_pad.shape[0]
    nseg16 = b_per_w // 16
    mesh = plsc.VectorSubcoreMesh(core_axis_name="c", subcore_axis_name="s")
    info = plsc.get_sparse_core_info()
    NC = info.num_cores

    @functools.partial(
        pl.kernel,
        mesh=mesh,
        out_type=jax.ShapeDtypeStruct((Bpad, 128), jnp.float32),
        scratch_types=[
            pltpu.VMEM((b_per_w,), jnp.int32),
            pltpu.VMEM((b_per_w, 128), jnp.float32),
            pltpu.SemaphoreType.DMA,
        ],
    )
    def gather_k(p_hbm, idx_hbm, out_hbm, idx_v, rows_v, sem):
        wid = lax.axis_index("s") * NC + lax.axis_index("c")
        base = wid * b_per_w
        pltpu.sync_copy(idx_hbm.at[pl.ds(base, b_per_w)], idx_v)
        for i in range(nseg16):
            sl = pl.ds(i * 16, 16)
            v = idx_v[sl]
            idx_v[sl] = jnp.bitwise_or(
                jnp.bitwise_and(v, n8 - 1),
                jnp.bitwise_and(lax.shift_right_logical(v, 3), n8))
        pltpu.async_copy(p_hbm.at[idx_v], rows_v, sem).wait()
        pltpu.sync_copy(rows_v, out_hbm.at[pl.ds(base, b_per_w)])

    return gather_k(Pp, idx_pad)


def _stage3_output(Gp, off, ptr_i, S, W3p, b3r, Bseg, n8_log2):
    D = W3p.shape[1]
    Bp1 = Bseg + 1

    def body(g_ref, off_ref, pi_ref, s_ref, w3_ref, b3_ref, o_ref):
        gp = g_ref[...][:Bp1, :]
        pi = pi_ref[...]
        rem = jnp.bitwise_and(
            lax.shift_right_logical(pi, n8_log2), 7)
        win = lax.broadcasted_iota(jnp.int32, (Bp1, 128), 1) // 16
        masked = jnp.where(win == rem, gp + off_ref[...], 0.0)
        ext = jnp.dot(masked, s_ref[...],
                      preferred_element_type=jnp.float32,
                      precision=lax.Precision.HIGHEST)
        d = ext[1:, :] - ext[:Bseg, :]
        cnt = (pi[1:, :] - pi[:Bseg, :]).astype(jnp.float32)
        o_ref[...] = (
            jnp.dot(d, w3_ref[...], preferred_element_type=jnp.float32,
                    precision=lax.Precision.HIGHEST)
            + cnt * b3_ref[...])

    return pl.pallas_call(
        body,
        out_shape=jax.ShapeDtypeStruct((Bseg, D), jnp.float32),
    )(Gp, off, ptr_i, S, W3p, b3r)


def kernel(dag_summaries, obs_ptr, W1, b1, W2, b2, W3, b3):
    N, D = dag_summaries.shape
    H1 = W1.shape[1]
    H2 = W2.shape[1]
    Bseg = obs_ptr.shape[0] - 1
    Rp = 1024
    N8 = N // 8
    n8_log2 = N8.bit_length() - 1

    ptr = obs_ptr.astype(jnp.int32)

    W2p = jnp.zeros((H1, 16), jnp.float32).at[:, :H2].set(W2)
    b2p = jnp.zeros((1, 16), jnp.float32).at[0, :H2].set(b2)
    W3p = jnp.zeros((16, D), jnp.float32).at[:H2, :].set(W3)
    b1r = b1.reshape(1, H1)
    b3r = b3.reshape(1, D)
    W2bd = jnp.kron(jnp.eye(8, dtype=jnp.float32), W2p)
    b2t = jnp.tile(b2p, (1, 8))

    tri, S, sred, wtri = _make_consts(Rp)
    Pp = _stage1_slab_prefix(dag_summaries, W1, b1r, W2bd, b2t,
                             tri, sred, wtri, Rp)

    info = plsc.get_sparse_core_info()
    NW = info.num_cores * info.num_subcores
    chunk = 16 * NW
    Bpad = ((Bseg + 1 + chunk - 1) // chunk) * chunk
    idx_pad = jnp.zeros((Bpad,), jnp.int32).at[:Bseg + 1].set(ptr)
    Gp = _stage2_gather(Pp, idx_pad, Bpad // NW, N8)

    off = lax.slice(Pp, (N8 + 1, 0), (N8 + 2, 128))
    ptr_i = ptr.reshape(Bseg + 1, 1)
    return _stage3_output(Gp, off, ptr_i, S, W3p, b3r, Bseg, n8_log2)


# speedup vs baseline: 2.1522x; 1.1319x over previous
import functools

import jax
import jax.numpy as jnp
from jax import lax
from jax.experimental import pallas as pl
from jax.experimental.pallas import tpu as pltpu
from jax.experimental.pallas import tpu_sc as plsc


def _stage1_slab_prefix(dag, W1, b1r, W2bd, b2t, tri, sred, wtri, Rp):
    N, D = dag.shape
    H1 = W1.shape[1]
    N8 = N // 8
    steps = N8 // Rp

    def body(x0, x1, x2, x3, x4, x5, x6, x7, w1_ref, b1_ref, w2_ref, b2_ref,
             tri_ref, sred_ref, wtri_ref, p_ref, carry_ref):
        g = pl.program_id(0)

        @pl.when(g == 0)
        def _():
            carry_ref[...] = jnp.zeros_like(carry_ref)

        carry = carry_ref[...]

        @pl.when(g < steps)
        def _():
            w1 = w1_ref[...]
            b1 = b1_ref[...]
            h1s = [
                jnp.maximum(
                    jnp.dot(x_ref[...], w1,
                            preferred_element_type=jnp.float32) + b1, 0.0)
                for x_ref in (x0, x1, x2, x3, x4, x5, x6, x7)
            ]
            h1cat = jnp.concatenate(h1s, axis=1)
            h2 = jnp.maximum(
                jnp.dot(h1cat, w2_ref[...],
                        preferred_element_type=jnp.float32) + b2_ref[...],
                0.0)
            tri = tri_ref[...]
            o = carry
            for c in range(Rp // 128):
                h2c = h2[c * 128:(c + 1) * 128, :]
                hl = jnp.dot(tri, h2c,
                             preferred_element_type=jnp.float32,
                             precision=lax.Precision.HIGHEST)
                p_ref[c * 128:(c + 1) * 128, :] = hl + o
                o = o + hl[127:128, :] + h2c[127:128, :]
            carry_ref[...] = o

        @pl.when(g == steps)
        def _():
            tot = jnp.dot(carry, sred_ref[...],
                          preferred_element_type=jnp.float32,
                          precision=lax.Precision.HIGHEST)
            off = jnp.dot(carry, wtri_ref[...],
                          preferred_element_type=jnp.float32,
                          precision=lax.Precision.HIGHEST)
            row = lax.broadcasted_iota(jnp.int32, (Rp, 128), 0)
            p_ref[...] = jnp.where(row == 0, tot,
                                   jnp.where(row == 1, off, 0.0))

    def xmap(k):
        return lambda g: (jnp.minimum(g, steps - 1) + k * steps, 0)

    return pl.pallas_call(
        body,
        grid=(steps + 1,),
        in_specs=[pl.BlockSpec((Rp, D), xmap(k)) for k in range(8)] + [
            pl.BlockSpec((D, H1), lambda g: (0, 0)),
            pl.BlockSpec((1, H1), lambda g: (0, 0)),
            pl.BlockSpec((128, 128), lambda g: (0, 0)),
            pl.BlockSpec((1, 128), lambda g: (0, 0)),
            pl.BlockSpec((128, 128), lambda g: (0, 0)),
            pl.BlockSpec((128, 128), lambda g: (0, 0)),
            pl.BlockSpec((128, 128), lambda g: (0, 0)),
        ],
        out_specs=pl.BlockSpec((Rp, 128), lambda g: (g, 0)),
        out_shape=jax.ShapeDtypeStruct((N8 + Rp, 128), jnp.float32),
        scratch_shapes=[pltpu.VMEM((1, 128), jnp.float32)],
    )(dag, dag, dag, dag, dag, dag, dag, dag,
      W1, b1r, W2bd, b2t, tri, sred, wtri)


def _make_consts(Rp):
    r2 = lax.broadcasted_iota(jnp.int32, (128, 128), 0)
    c2 = lax.broadcasted_iota(jnp.int32, (128, 128), 1)
    S = (r2 % 16 == c2).astype(jnp.float32)[:, :16]
    sred = (r2 % 16 == c2 % 16).astype(jnp.float32)
    wtri = ((r2 % 16 == c2 % 16) & (r2 // 16 < c2 // 16)).astype(jnp.float32)
    tri = (c2 < r2).astype(jnp.float32)
    return tri, S, sred, wtri


def _stage2_gather(Pp, idx_pad, b_per_w, n8):
    Bpad = idx_pad.shape[0]
    nseg16 = b_per_w // 16
    mesh = plsc.VectorSubcoreMesh(core_axis_name="c", subcore_axis_name="s")
    info = plsc.get_sparse_core_info()
    NC = info.num_cores

    @functools.partial(
        pl.kernel,
        mesh=mesh,
        out_type=jax.ShapeDtypeStruct((Bpad, 128), jnp.float32),
        scratch_types=[
            pltpu.VMEM((b_per_w,), jnp.int32),
            pltpu.VMEM((b_per_w, 128), jnp.float32),
            pltpu.SemaphoreType.DMA,
        ],
    )
    def gather_k(p_hbm, idx_hbm, out_hbm, idx_v, rows_v, sem):
        wid = lax.axis_index("s") * NC + lax.axis_index("c")
        base = wid * b_per_w
        pltpu.sync_copy(idx_hbm.at[pl.ds(base, b_per_w)], idx_v)
        for i in range(nseg16):
            sl = pl.ds(i * 16, 16)
            v = idx_v[sl]
            idx_v[sl] = jnp.bitwise_or(
                jnp.bitwise_and(v, n8 - 1),
                jnp.bitwise_and(lax.shift_right_logical(v, 3), n8))
        pltpu.async_copy(p_hbm.at[idx_v], rows_v, sem).wait()
        pltpu.sync_copy(rows_v, out_hbm.at[pl.ds(base, b_per_w)])

    return gather_k(Pp, idx_pad)


def _stage3_output(Gp, off, ptr_i, S, W3p, b3r, Bseg, n8_log2):
    D = W3p.shape[1]
    Bp1 = Bseg + 1

    def body(g_ref, off_ref, pi_ref, s_ref, w3_ref, b3_ref, o_ref):
        gp = g_ref[...][:Bp1, :]
        pi = pi_ref[...]
        rem = jnp.bitwise_and(
            lax.shift_right_logical(pi, n8_log2), 7)
        win = lax.broadcasted_iota(jnp.int32, (Bp1, 128), 1) // 16
        masked = jnp.where(win == rem, gp + off_ref[...], 0.0)
        ext = jnp.dot(masked, s_ref[...],
                      preferred_element_type=jnp.float32,
                      precision=lax.Precision.HIGHEST)
        d = ext[1:, :] - ext[:Bseg, :]
        cnt = (pi[1:, :] - pi[:Bseg, :]).astype(jnp.float32)
        o_ref[...] = (
            jnp.dot(d, w3_ref[...], preferred_element_type=jnp.float32,
                    precision=lax.Precision.HIGHEST)
            + cnt * b3_ref[...])

    return pl.pallas_call(
        body,
        out_shape=jax.ShapeDtypeStruct((Bseg, D), jnp.float32),
    )(Gp, off, ptr_i, S, W3p, b3r)


def kernel(dag_summaries, obs_ptr, W1, b1, W2, b2, W3, b3):
    N, D = dag_summaries.shape
    H1 = W1.shape[1]
    H2 = W2.shape[1]
    Bseg = obs_ptr.shape[0] - 1
    Rp = 2048
    N8 = N // 8
    n8_log2 = N8.bit_length() - 1

    ptr = obs_ptr.astype(jnp.int32)

    W2p = jnp.zeros((H1, 16), jnp.float32).at[:, :H2].set(W2)
    b2p = jnp.zeros((1, 16), jnp.float32).at[0, :H2].set(b2)
    W3p = jnp.zeros((16, D), jnp.float32).at[:H2, :].set(W3)
    b1r = b1.reshape(1, H1)
    b3r = b3.reshape(1, D)
    W2bd = jnp.kron(jnp.eye(8, dtype=jnp.float32), W2p)
    b2t = jnp.tile(b2p, (1, 8))

    tri, S, sred, wtri = _make_consts(Rp)
    Pp = _stage1_slab_prefix(dag_summaries, W1, b1r, W2bd, b2t,
                             tri, sred, wtri, Rp)

    info = plsc.get_sparse_core_info()
    NW = info.num_cores * info.num_subcores
    chunk = 16 * NW
    Bpad = ((Bseg + 1 + chunk - 1) // chunk) * chunk
    idx_pad = jnp.zeros((Bpad,), jnp.int32).at[:Bseg + 1].set(ptr)
    Gp = _stage2_gather(Pp, idx_pad, Bpad // NW, N8)

    off = lax.slice(Pp, (N8 + 1, 0), (N8 + 2, 128))
    ptr_i = ptr.reshape(Bseg + 1, 1)
    return _stage3_output(Gp, off, ptr_i, S, W3p, b3r, Bseg, n8_log2)


# speedup vs baseline: 2.2424x; 1.0419x over previous
import functools

import jax
import jax.numpy as jnp
from jax import lax
from jax.experimental import pallas as pl
from jax.experimental.pallas import tpu as pltpu
from jax.experimental.pallas import tpu_sc as plsc


def _stage1_slab_prefix(dag, W1, b1r, W2bd, b2t, tri, sred, wtri, Rp):
    N, D = dag.shape
    H1 = W1.shape[1]
    N8 = N // 8
    steps = N8 // Rp

    def body(x0, x1, x2, x3, x4, x5, x6, x7, w1_ref, b1_ref, w2_ref, b2_ref,
             tri_ref, sred_ref, wtri_ref, p_ref, carry_ref):
        g = pl.program_id(0)

        @pl.when(g == 0)
        def _():
            carry_ref[...] = jnp.zeros_like(carry_ref)

        carry = carry_ref[...]

        @pl.when(g < steps)
        def _():
            w1 = w1_ref[...]
            b1 = b1_ref[...]
            h1s = [
                jnp.maximum(
                    jnp.dot(x_ref[...], w1,
                            preferred_element_type=jnp.float32) + b1, 0.0)
                for x_ref in (x0, x1, x2, x3, x4, x5, x6, x7)
            ]
            h1cat = jnp.concatenate(h1s, axis=1)
            h2 = jnp.maximum(
                jnp.dot(h1cat, w2_ref[...],
                        preferred_element_type=jnp.float32) + b2_ref[...],
                0.0)
            tri = tri_ref[...]
            o = carry
            for c in range(Rp // 128):
                h2c = h2[c * 128:(c + 1) * 128, :]
                hl = jnp.dot(tri, h2c,
                             preferred_element_type=jnp.float32,
                             precision=lax.Precision.HIGHEST)
                p_ref[c * 128:(c + 1) * 128, :] = hl + o
                o = o + hl[127:128, :] + h2c[127:128, :]
            carry_ref[...] = o

        @pl.when(g == steps)
        def _():
            tot = jnp.dot(carry, sred_ref[...],
                          preferred_element_type=jnp.float32,
                          precision=lax.Precision.HIGHEST)
            off = jnp.dot(carry, wtri_ref[...],
                          preferred_element_type=jnp.float32,
                          precision=lax.Precision.HIGHEST)
            row = lax.broadcasted_iota(jnp.int32, (Rp, 128), 0)
            p_ref[...] = jnp.where(row == 0, tot,
                                   jnp.where(row == 1, off, 0.0))

    def xmap(k):
        return lambda g: (jnp.minimum(g, steps - 1) + k * steps, 0)

    return pl.pallas_call(
        body,
        grid=(steps + 1,),
        in_specs=[pl.BlockSpec((Rp, D), xmap(k)) for k in range(8)] + [
            pl.BlockSpec((D, H1), lambda g: (0, 0)),
            pl.BlockSpec((1, H1), lambda g: (0, 0)),
            pl.BlockSpec((128, 128), lambda g: (0, 0)),
            pl.BlockSpec((1, 128), lambda g: (0, 0)),
            pl.BlockSpec((128, 128), lambda g: (0, 0)),
            pl.BlockSpec((128, 128), lambda g: (0, 0)),
            pl.BlockSpec((128, 128), lambda g: (0, 0)),
        ],
        out_specs=pl.BlockSpec((Rp, 128), lambda g: (g, 0)),
        out_shape=jax.ShapeDtypeStruct((N8 + Rp, 128), jnp.float32),
        scratch_shapes=[pltpu.VMEM((1, 128), jnp.float32)],
    )(dag, dag, dag, dag, dag, dag, dag, dag,
      W1, b1r, W2bd, b2t, tri, sred, wtri)


def _make_consts(Rp):
    r2 = lax.broadcasted_iota(jnp.int32, (128, 128), 0)
    c2 = lax.broadcasted_iota(jnp.int32, (128, 128), 1)
    S = (r2 % 16 == c2).astype(jnp.float32)[:, :16]
    sred = (r2 % 16 == c2 % 16).astype(jnp.float32)
    wtri = ((r2 % 16 == c2 % 16) & (r2 // 16 < c2 // 16)).astype(jnp.float32)
    tri = (c2 < r2).astype(jnp.float32)
    return tri, S, sred, wtri


def _stage2_gather(Pp, idx_pad, b_per_w, n8):
    Bpad = idx_pad.shape[0]
    nseg16 = b_per_w // 16
    mesh = plsc.VectorSubcoreMesh(core_axis_name="c", subcore_axis_name="s")
    info = plsc.get_sparse_core_info()
    NC = info.num_cores

    @functools.partial(
        pl.kernel,
        mesh=mesh,
        out_type=jax.ShapeDtypeStruct((Bpad, 128), jnp.float32),
        scratch_types=[
            pltpu.VMEM((b_per_w,), jnp.int32),
            pltpu.VMEM((b_per_w, 128), jnp.float32),
            pltpu.SemaphoreType.DMA,
        ],
    )
    def gather_k(p_hbm, idx_hbm, out_hbm, idx_v, rows_v, sem):
        wid = lax.axis_index("s") * NC + lax.axis_index("c")
        base = wid * b_per_w
        pltpu.sync_copy(idx_hbm.at[pl.ds(base, b_per_w)], idx_v)
        for i in range(nseg16):
            sl = pl.ds(i * 16, 16)
            v = idx_v[sl]
            idx_v[sl] = jnp.bitwise_or(
                jnp.bitwise_and(v, n8 - 1),
                jnp.bitwise_and(lax.shift_right_logical(v, 3), n8))
        pltpu.async_copy(p_hbm.at[idx_v], rows_v, sem).wait()
        pltpu.sync_copy(rows_v, out_hbm.at[pl.ds(base, b_per_w)])

    return gather_k(Pp, idx_pad)


def _stage3_output(Gp, off, ptr_i, S, W3p, b3r, Bseg, n8_log2):
    D = W3p.shape[1]
    Bp1 = Bseg + 1

    def body(g_ref, off_ref, pi_ref, s_ref, w3_ref, b3_ref, o_ref):
        gp = g_ref[...][:Bp1, :]
        pi = pi_ref[...]
        rem = jnp.bitwise_and(
            lax.shift_right_logical(pi, n8_log2), 7)
        win = lax.broadcasted_iota(jnp.int32, (Bp1, 128), 1) // 16
        masked = jnp.where(win == rem, gp + off_ref[...], 0.0)
        ext = jnp.dot(masked, s_ref[...],
                      preferred_element_type=jnp.float32,
                      precision=lax.Precision.HIGHEST)
        d = ext[1:, :] - ext[:Bseg, :]
        cnt = (pi[1:, :] - pi[:Bseg, :]).astype(jnp.float32)
        o_ref[...] = (
            jnp.dot(d, w3_ref[...], preferred_element_type=jnp.float32,
                    precision=lax.Precision.HIGHEST)
            + cnt * b3_ref[...])

    return pl.pallas_call(
        body,
        out_shape=jax.ShapeDtypeStruct((Bseg, D), jnp.float32),
    )(Gp, off, ptr_i, S, W3p, b3r)


def kernel(dag_summaries, obs_ptr, W1, b1, W2, b2, W3, b3):
    N, D = dag_summaries.shape
    H1 = W1.shape[1]
    H2 = W2.shape[1]
    Bseg = obs_ptr.shape[0] - 1
    Rp = 4096
    N8 = N // 8
    n8_log2 = N8.bit_length() - 1

    ptr = obs_ptr.astype(jnp.int32)

    W2p = jnp.zeros((H1, 16), jnp.float32).at[:, :H2].set(W2)
    b2p = jnp.zeros((1, 16), jnp.float32).at[0, :H2].set(b2)
    W3p = jnp.zeros((16, D), jnp.float32).at[:H2, :].set(W3)
    b1r = b1.reshape(1, H1)
    b3r = b3.reshape(1, D)
    W2bd = jnp.kron(jnp.eye(8, dtype=jnp.float32), W2p)
    b2t = jnp.tile(b2p, (1, 8))

    tri, S, sred, wtri = _make_consts(Rp)
    Pp = _stage1_slab_prefix(dag_summaries, W1, b1r, W2bd, b2t,
                             tri, sred, wtri, Rp)

    info = plsc.get_sparse_core_info()
    NW = info.num_cores * info.num_subcores
    chunk = 16 * NW
    Bpad = ((Bseg + 1 + chunk - 1) // chunk) * chunk
    idx_pad = jnp.zeros((Bpad,), jnp.int32).at[:Bseg + 1].set(ptr)
    Gp = _stage2_gather(Pp, idx_pad, Bpad // NW, N8)

    off = lax.slice(Pp, (N8 + 1, 0), (N8 + 2, 128))
    ptr_i = ptr.reshape(Bseg + 1, 1)
    return _stage3_output(Gp, off, ptr_i, S, W3p, b3r, Bseg, n8_log2)
